# baseline (speedup 1.0000x reference)
import functools

import jax
import jax.numpy as jnp
from jax import lax
from jax.experimental import pallas as pl
from jax.experimental.pallas import tpu as pltpu
from jax.experimental.pallas import tpu_sc as plsc

_NC = 2
_NS = 16
_NW = _NC * _NS
_BBLK = 128


def _sc_gather_t(table, xt):
    hist, batch = xt.shape
    d_model = table.shape[1]
    mesh = plsc.VectorSubcoreMesh(core_axis_name="c", subcore_axis_name="s")

    @functools.partial(
        pl.kernel,
        mesh=mesh,
        out_type=jax.ShapeDtypeStruct((hist, d_model, batch), jnp.float32),
        scratch_types=[
            pltpu.VMEM((hist, _BBLK), jnp.int32),
            pltpu.VMEM((2, _BBLK, d_model), jnp.float32),
            pltpu.VMEM((2, d_model, _BBLK), jnp.float32),
            pltpu.SemaphoreType.DMA((2,)),
            pltpu.SemaphoreType.DMA((2,)),
        ],
        compiler_params=pltpu.CompilerParams(use_tc_tiling_on_sc=False,
                                             needs_layout_passes=False),
    )
    def k(table_hbm, xt_hbm, out_hbm, idx_v, rows_v, rows_t, gsem, ssem):
        wid = lax.axis_index("s") * _NC + lax.axis_index("c")
        col0 = wid * _BBLK
        pltpu.sync_copy(xt_hbm.at[:, pl.ds(col0, _BBLK)], idx_v)

        def fire_gather(h, s):
            pltpu.async_copy(table_hbm.at[idx_v.at[h]], rows_v.at[s],
                             gsem.at[s])

        def drain_gather(s):
            pltpu.make_async_copy(table_hbm.at[idx_v.at[0]], rows_v.at[s],
                                  gsem.at[s]).wait()

        def fire_store(h, s):
            pltpu.async_copy(rows_t.at[s],
                             out_hbm.at[h, :, pl.ds(col0, _BBLK)],
                             ssem.at[s])

        def drain_store(s):
            pltpu.make_async_copy(rows_t.at[s],
                                  out_hbm.at[0, :, pl.ds(col0, _BBLK)],
                                  ssem.at[s]).wait()

        iota = lax.iota(jnp.int32, 16)

        def transpose(s):
            for d in range(d_model):
                cd = jnp.full((16,), d, jnp.int32)
                for gb in range(_BBLK // 16):
                    v = plsc.load_gather(rows_v.at[s], [gb * 16 + iota, cd])
                    rows_t[s, d, pl.ds(gb * 16, 16)] = v

        fire_gather(0, 0)

        def outer(i, carry):
            for p in range(2):
                h = i * 2 + p
                cur, nxt = p, 1 - p

                @pl.when(h + 1 < hist)
                def _():
                    fire_gather(h + 1, nxt)

                drain_gather(cur)

                @pl.when(h >= 2)
                def _():
                    drain_store(cur)

                transpose(cur)
                fire_store(h, cur)
            return carry

        lax.fori_loop(0, hist // 2, outer, 0)
        drain_store(0)
        drain_store(1)

    return k(table, xt)


def kernel(x, weights):
    xt = jnp.transpose(x).astype(jnp.int32)
    out_t = _sc_gather_t(weights, xt)
    return jnp.transpose(out_t, (2, 0, 1))


# speedup vs baseline: 1.3230x; 1.3230x over previous
import functools

import jax
import jax.numpy as jnp
from jax import lax
from jax.experimental import pallas as pl
from jax.experimental.pallas import tpu as pltpu
from jax.experimental.pallas import tpu_sc as plsc

_NC = 2
_NS = 16
_NW = _NC * _NS
_BBLK = 128


def _sc_gather_t(table, xt):
    hist, batch = xt.shape
    d_model = table.shape[1]
    mesh = plsc.VectorSubcoreMesh(core_axis_name="c", subcore_axis_name="s")

    @functools.partial(
        pl.kernel,
        mesh=mesh,
        out_type=jax.ShapeDtypeStruct((hist, d_model, batch), jnp.float32),
        scratch_types=[
            pltpu.VMEM((hist, _BBLK), jnp.int32),
            pltpu.VMEM((2, _BBLK, d_model), jnp.float32),
            pltpu.VMEM((2, d_model, _BBLK), jnp.float32),
            pltpu.SemaphoreType.DMA((2,)),
            pltpu.SemaphoreType.DMA((2,)),
        ],
        compiler_params=pltpu.CompilerParams(use_tc_tiling_on_sc=False,
                                             needs_layout_passes=False),
    )
    def k(table_hbm, xt_hbm, out_hbm, idx_v, rows_v, rows_t, gsem, ssem):
        wid = lax.axis_index("s") * _NC + lax.axis_index("c")
        col0 = wid * _BBLK
        pltpu.sync_copy(xt_hbm.at[:, pl.ds(col0, _BBLK)], idx_v)

        def fire_gather(h, s):
            pltpu.async_copy(table_hbm.at[idx_v.at[h]], rows_v.at[s],
                             gsem.at[s])

        def drain_gather(s):
            pltpu.make_async_copy(table_hbm.at[idx_v.at[0]], rows_v.at[s],
                                  gsem.at[s]).wait()

        def fire_store(h, s):
            pltpu.async_copy(rows_t.at[s],
                             out_hbm.at[h, :, pl.ds(col0, _BBLK)],
                             ssem.at[s])

        def drain_store(s):
            pltpu.make_async_copy(rows_t.at[s],
                                  out_hbm.at[0, :, pl.ds(col0, _BBLK)],
                                  ssem.at[s]).wait()

        iota = lax.iota(jnp.int32, 16)

        def transpose(s):
            @plsc.parallel_loop(0, _BBLK // 16, unroll=2)
            def _(gb):
                ri = gb * 16 + iota
                for d in range(d_model):
                    cd = jnp.full((16,), d, jnp.int32)
                    v = plsc.load_gather(rows_v.at[s], [ri, cd])
                    rows_t[s, d, pl.ds(gb * 16, 16)] = v

        fire_gather(0, 0)

        def outer(i, carry):
            for p in range(2):
                h = i * 2 + p
                cur, nxt = p, 1 - p

                @pl.when(h + 1 < hist)
                def _():
                    fire_gather(h + 1, nxt)

                drain_gather(cur)

                @pl.when(h >= 2)
                def _():
                    drain_store(cur)

                transpose(cur)
                fire_store(h, cur)
            return carry

        lax.fori_loop(0, hist // 2, outer, 0)
        drain_store(0)
        drain_store(1)

    return k(table, xt)


def kernel(x, weights):
    xt = jnp.transpose(x).astype(jnp.int32)
    out_t = _sc_gather_t(weights, xt)
    return jnp.transpose(out_t, (2, 0, 1))


# speedup vs baseline: 1.6364x; 1.2369x over previous
import functools

import jax
import jax.numpy as jnp
from jax import lax
from jax.experimental import pallas as pl
from jax.experimental.pallas import tpu as pltpu
from jax.experimental.pallas import tpu_sc as plsc

_NC = 2
_NS = 16
_NW = _NC * _NS
_CHUNK = 128
_GK = 4
_NSET = 2
_GKC = _GK * _CHUNK


def _sc_gather(table, idx3, n_chunks):
    d_model = table.shape[1]
    n_groups = n_chunks // _GK
    n_outer = n_groups // _NSET
    per_w = n_chunks * _CHUNK
    mesh = plsc.VectorSubcoreMesh(core_axis_name="c", subcore_axis_name="s")

    @functools.partial(
        pl.kernel,
        mesh=mesh,
        out_type=jax.ShapeDtypeStruct((_NW * per_w, d_model), jnp.float32),
        scratch_types=[
            pltpu.VMEM((n_chunks, _CHUNK), jnp.int32),
            pltpu.VMEM((_NSET, _GKC, d_model), jnp.float32),
            pltpu.SemaphoreType.DMA((_NSET,)),
            pltpu.SemaphoreType.DMA((_NSET,)),
        ],
        compiler_params=pltpu.CompilerParams(use_tc_tiling_on_sc=False),
    )
    def k(table_hbm, idx_hbm, out_hbm, idx_v, rows_v, gsem, ssem):
        wid = lax.axis_index("s") * _NC + lax.axis_index("c")
        pltpu.sync_copy(idx_hbm.at[wid], idx_v)
        base = wid * per_w

        def fire_gathers(g, s):
            for b in range(_GK):
                pltpu.async_copy(
                    table_hbm.at[idx_v.at[g * _GK + b]],
                    rows_v.at[s, pl.ds(b * _CHUNK, _CHUNK)],
                    gsem.at[s])

        def drain_gathers(s):
            for b in range(_GK):
                pltpu.make_async_copy(
                    table_hbm.at[idx_v.at[0]],
                    rows_v.at[s, pl.ds(b * _CHUNK, _CHUNK)],
                    gsem.at[s]).wait()

        def fire_store(g, s):
            pltpu.async_copy(rows_v.at[s],
                             out_hbm.at[pl.ds(base + g * _GKC, _GKC)],
                             ssem.at[s])

        def drain_store(s):
            pltpu.make_async_copy(rows_v.at[s],
                                  out_hbm.at[pl.ds(base, _GKC)],
                                  ssem.at[s]).wait()

        fire_gathers(0, 0)

        def outer(i, carry):
            for p in range(_NSET):
                g = i * _NSET + p
                cur = p
                nxt = (p + 1) % _NSET
                @pl.when(g + 1 - _NSET >= 0)
                def _():
                    drain_store(nxt)

                @pl.when(g + 1 < n_groups)
                def _():
                    fire_gathers(g + 1, nxt)

                drain_gathers(cur)
                fire_store(g, cur)
            return carry

        lax.fori_loop(0, n_outer, outer, 0)
        drain_store((n_groups - 1) % _NSET)

    return k(table, idx3)


def _tc_relayout(x3):
    batch, hist, d_model = x3.shape
    bblk, hblk = 512, 8

    def body(x_ref, o_ref):
        v = x_ref[...].reshape(bblk, hblk * d_model)
        o_ref[...] = jnp.transpose(v).reshape(hblk, d_model, bblk)

    return pl.pallas_call(
        body,
        grid=(batch // bblk, hist // hblk),
        in_specs=[pl.BlockSpec((bblk, hblk, d_model),
                               lambda i, j: (i, j, 0))],
        out_specs=pl.BlockSpec((hblk, d_model, bblk),
                               lambda i, j: (j, 0, i)),
        out_shape=jax.ShapeDtypeStruct((hist, d_model, batch), jnp.float32),
    )(x3)


def kernel(x, weights):
    batch, hist = x.shape
    d_model = weights.shape[1]
    total = batch * hist
    n_chunks = total // (_NW * _CHUNK)
    idx3 = x.reshape(_NW, n_chunks, _CHUNK).astype(jnp.int32)
    rows = _sc_gather(weights, idx3, n_chunks)
    out_t = _tc_relayout(rows.reshape(batch, hist, d_model))
    return jnp.transpose(out_t, (2, 0, 1))


# speedup vs baseline: 2.2023x; 1.3458x over previous
import functools

import jax
import jax.numpy as jnp
from jax import lax
from jax.experimental import pallas as pl
from jax.experimental.pallas import tpu as pltpu
from jax.experimental.pallas import tpu_sc as plsc

_NC = 2
_NS = 16
_NW = _NC * _NS
_BBLK = 128
_PITCH = 129


def _sc_gather_t(table, xt):
    hist, batch = xt.shape
    d_model = table.shape[1]
    mesh = plsc.VectorSubcoreMesh(core_axis_name="c", subcore_axis_name="s")

    @functools.partial(
        pl.kernel,
        mesh=mesh,
        out_type=jax.ShapeDtypeStruct((hist, d_model, batch), jnp.float32),
        scratch_types=[
            pltpu.VMEM((hist, _BBLK), jnp.int32),
            pltpu.VMEM((2, _BBLK, d_model), jnp.float32),
            pltpu.VMEM((2, d_model, _PITCH), jnp.float32),
            pltpu.SemaphoreType.DMA((2,)),
            pltpu.SemaphoreType.DMA((2,)),
        ],
        compiler_params=pltpu.CompilerParams(use_tc_tiling_on_sc=False,
                                             needs_layout_passes=False),
    )
    def k(table_hbm, xt_hbm, out_hbm, idx_v, rows_v, rows_t, gsem, ssem):
        wid = lax.axis_index("s") * _NC + lax.axis_index("c")
        col0 = wid * _BBLK
        pltpu.sync_copy(xt_hbm.at[:, pl.ds(col0, _BBLK)], idx_v)

        def fire_gather(h, s):
            pltpu.async_copy(table_hbm.at[idx_v.at[h]], rows_v.at[s],
                             gsem.at[s])

        def drain_gather(s):
            pltpu.make_async_copy(table_hbm.at[idx_v.at[0]], rows_v.at[s],
                                  gsem.at[s]).wait()

        def fire_store(h, s):
            pltpu.async_copy(rows_t.at[s, :, pl.ds(0, _BBLK)],
                             out_hbm.at[h, :, pl.ds(col0, _BBLK)],
                             ssem.at[s])

        def drain_store(s):
            pltpu.make_async_copy(rows_t.at[s, :, pl.ds(0, _BBLK)],
                                  out_hbm.at[0, :, pl.ds(col0, _BBLK)],
                                  ssem.at[s]).wait()

        iota = lax.iota(jnp.int32, 16)

        def transpose(s):
            @plsc.parallel_loop(0, _BBLK, unroll=8)
            def _(b):
                cb = jnp.full((16,), 0, jnp.int32) + b
                for dc in range(d_model // 16):
                    v = rows_v[s, b, pl.ds(dc * 16, 16)]
                    plsc.store_scatter(rows_t.at[s],
                                       [dc * 16 + iota, cb], v)

        fire_gather(0, 0)

        def outer(i, carry):
            for p in range(2):
                h = i * 2 + p
                cur, nxt = p, 1 - p

                @pl.when(h + 1 < hist)
                def _():
                    fire_gather(h + 1, nxt)

                drain_gather(cur)

                @pl.when(h >= 2)
                def _():
                    drain_store(cur)

                transpose(cur)
                fire_store(h, cur)
            return carry

        lax.fori_loop(0, hist // 2, outer, 0)
        drain_store(0)
        drain_store(1)

    return k(table, xt)


def kernel(x, weights):
    xt = jnp.transpose(x).astype(jnp.int32)
    out_t = _sc_gather_t(weights, xt)
    return jnp.transpose(out_t, (2, 0, 1))
